# baseline (speedup 1.0000x reference)
import functools

import jax
import jax.numpy as jnp
from jax import lax
from jax.experimental import pallas as pl
from jax.experimental.pallas import tpu as pltpu
from jax.experimental.pallas import tpu_sc as plsc

_N_ROWS = 4096
_N_COLS = 50
_B = _N_ROWS * _N_COLS
_D = 128
_BQ = _B // 4
_NC = 2
_NS = 16
_NW = _NC * _NS
_RPW = _N_ROWS // _NW
_CR = 8
_CH = _CR * _N_COLS // 4
_BPW = _BQ // _NW
_NCHUNK = _RPW // _CR
_NBUF = 2
_CHP = 104
_IPW = _NCHUNK * _CHP

_mesh = plsc.VectorSubcoreMesh(core_axis_name="c", subcore_axis_name="s")


@functools.partial(
    pl.kernel,
    mesh=_mesh,
    compiler_params=pltpu.CompilerParams(use_tc_tiling_on_sc=True),
    out_type=jax.ShapeDtypeStruct((_N_ROWS, _N_COLS, _D), jnp.float32),
    scratch_types=[
        pltpu.VMEM((_IPW,), jnp.int32),
        pltpu.VMEM((_NBUF, _CH, 4, _D), jnp.float32),
        pltpu.SemaphoreType.DMA,
        pltpu.SemaphoreType.DMA,
        pltpu.SemaphoreType.DMA,
        pltpu.SemaphoreType.DMA,
    ],
)
def _embed_sc(qidx_hbm, qtable_hbm, out_hbm, idx_v, rows_q, g0, g1, o0, o1):
    rows_v = rows_q.reshape(_NBUF, _CR, _N_COLS, _D)
    wid = lax.axis_index("s") * _NC + lax.axis_index("c")
    rbase = wid * _RPW
    pltpu.sync_copy(qidx_hbm.at[pl.ds(wid * _IPW, _IPW)], idx_v)

    gsems = (g0, g1)
    osems = (o0, o1)

    def start_gather(c):
        b = c % _NBUF
        return pltpu.async_copy(
            qtable_hbm.at[idx_v.at[pl.ds(c * _CHP, _CH)]], rows_q.at[b],
            gsems[b])

    gather_pending = [None] * _NBUF
    out_pending = [None] * _NBUF
    gather_pending[0] = start_gather(0)
    for c in range(_NCHUNK):
        b = c % _NBUF
        gather_pending[b].wait()
        out_pending[b] = pltpu.async_copy(
            rows_v.at[b], out_hbm.at[pl.ds(rbase + c * _CR, _CR)], osems[b])
        n = c + 1
        if n < _NCHUNK:
            bn = n % _NBUF
            if out_pending[bn] is not None:
                out_pending[bn].wait()
                out_pending[bn] = None
            gather_pending[bn] = start_gather(n)
    for b in range(_NBUF):
        if out_pending[b] is not None:
            out_pending[b].wait()


def kernel(cue, table):
    idx = cue.reshape(_BQ, 4).astype(jnp.int32)
    qidx = ((idx[:, 0] * 4 + idx[:, 1]) * 4 + idx[:, 2]) * 4 + idx[:, 3]
    qidx = jnp.pad(qidx.reshape(_NW, _NCHUNK, _CH),
                   ((0, 0), (0, 0), (0, _CHP - _CH))).reshape(-1)
    digits = (jnp.arange(256, dtype=jnp.int32)[:, None]
              // jnp.array([64, 16, 4, 1], dtype=jnp.int32)) % 4
    qtable = jnp.take(table.astype(jnp.float32), digits, axis=0)
    return _embed_sc(qidx, qtable)


# speedup vs baseline: 1.8597x; 1.8597x over previous
import functools

import jax
import jax.numpy as jnp
from jax import lax
from jax.experimental import pallas as pl
from jax.experimental.pallas import tpu as pltpu
from jax.experimental.pallas import tpu_sc as plsc

_N_ROWS = 4096
_N_COLS = 50
_D = 128
_NC = 2
_NS = 16
_NW = _NC * _NS
_IPW = _N_ROWS // _NW
_QPW = _IPW * _N_COLS // 4
_NVEC = _QPW // 16
_IQ = _IPW // 4
_CJ = 2
_CH = _CJ * _IQ
_NCHUNK = _N_COLS // _CJ
_NBUF = 2

_mesh = plsc.VectorSubcoreMesh(core_axis_name="c", subcore_axis_name="s")


@functools.partial(
    pl.kernel,
    mesh=_mesh,
    compiler_params=pltpu.CompilerParams(needs_layout_passes=False),
    out_type=jax.ShapeDtypeStruct((_N_COLS, _N_ROWS, _D), jnp.float32),
    scratch_types=[
        pltpu.VMEM((_IPW, _N_COLS), jnp.int32),
        pltpu.VMEM((_QPW,), jnp.int32),
        pltpu.VMEM((_NBUF, _CH, 4, _D), jnp.float32),
        pltpu.SemaphoreType.DMA,
        pltpu.SemaphoreType.DMA,
        pltpu.SemaphoreType.DMA,
        pltpu.SemaphoreType.DMA,
    ],
)
def _embed_sc(cue_hbm, qtable_hbm, out_hbm, cue_v, idx_v, rows_q,
              g0, g1, o0, o1):
    rows_t = rows_q.reshape(_NBUF, _CJ, _IPW, _D)
    wid = lax.axis_index("s") * _NC + lax.axis_index("c")
    i0 = wid * _IPW
    pltpu.sync_copy(cue_hbm.at[pl.ds(i0, _IPW)], cue_v)

    lanes = lax.iota(jnp.int32, 16)

    def pack(v, carry):
        t = v * 16 + lanes
        col = lax.shift_right_logical(t, 5)
        row4 = lax.shift_left(lax.bitwise_and(t, _IQ - 1), 2)
        g0_ = plsc.load_gather(cue_v, [row4, col])
        g1_ = plsc.load_gather(cue_v, [row4 + 1, col])
        g2_ = plsc.load_gather(cue_v, [row4 + 2, col])
        g3_ = plsc.load_gather(cue_v, [row4 + 3, col])
        q = ((g0_ * 4 + g1_) * 4 + g2_) * 4 + g3_
        idx_v[pl.ds(pl.multiple_of(v * 16, 16), 16)] = q
        return carry

    lax.fori_loop(0, _NVEC, pack, 0)

    gsems = (g0, g1)
    osems = (o0, o1)

    def start_gather(c):
        b = c % _NBUF
        return pltpu.async_copy(
            qtable_hbm.at[idx_v.at[pl.ds(c * _CH, _CH)]], rows_q.at[b],
            gsems[b])

    gather_pending = [None] * _NBUF
    out_pending = [None] * _NBUF
    gather_pending[0] = start_gather(0)
    for c in range(_NCHUNK):
        b = c % _NBUF
        gather_pending[b].wait()
        out_pending[b] = pltpu.async_copy(
            rows_t.at[b],
            out_hbm.at[pl.ds(c * _CJ, _CJ), pl.ds(i0, _IPW)], osems[b])
        n = c + 1
        if n < _NCHUNK:
            bn = n % _NBUF
            if out_pending[bn] is not None:
                out_pending[bn].wait()
                out_pending[bn] = None
            gather_pending[bn] = start_gather(n)
    for b in range(_NBUF):
        if out_pending[b] is not None:
            out_pending[b].wait()


def kernel(cue, table):
    digits = (jnp.arange(256, dtype=jnp.int32)[:, None]
              // jnp.array([64, 16, 4, 1], dtype=jnp.int32)) % 4
    qtable = jnp.take(table.astype(jnp.float32), digits, axis=0)
    out_t = _embed_sc(cue.astype(jnp.int32), qtable)
    return jnp.transpose(out_t, (1, 0, 2))


# speedup vs baseline: 1.8735x; 1.0074x over previous
import functools

import jax
import jax.numpy as jnp
from jax import lax
from jax.experimental import pallas as pl
from jax.experimental.pallas import tpu as pltpu
from jax.experimental.pallas import tpu_sc as plsc

_N_ROWS = 4096
_N_COLS = 50
_D = 128
_NC = 2
_NS = 16
_NW = _NC * _NS
_IPW = _N_ROWS // _NW
_QPW = _IPW * _N_COLS // 4
_NVEC = _QPW // 16
_IQ = _IPW // 4
_CJ = 2
_CH = _CJ * _IQ
_NCHUNK = _N_COLS // _CJ
_NBUF = 3

_mesh = plsc.VectorSubcoreMesh(core_axis_name="c", subcore_axis_name="s")


@functools.partial(
    pl.kernel,
    mesh=_mesh,
    compiler_params=pltpu.CompilerParams(needs_layout_passes=False),
    out_type=jax.ShapeDtypeStruct((_N_COLS, _N_ROWS, _D), jnp.float32),
    scratch_types=[
        pltpu.VMEM((_IPW, _N_COLS), jnp.int32),
        pltpu.VMEM((_QPW,), jnp.int32),
        pltpu.VMEM((_NBUF, _CH, 4, _D), jnp.float32),
        pltpu.SemaphoreType.DMA,
        pltpu.SemaphoreType.DMA,
        pltpu.SemaphoreType.DMA,
        pltpu.SemaphoreType.DMA,
        pltpu.SemaphoreType.DMA,
        pltpu.SemaphoreType.DMA,
    ],
)
def _embed_sc(cue_hbm, qtable_hbm, out_hbm, cue_v, idx_v, rows_q,
              g0, g1, g2, o0, o1, o2):
    rows_t = rows_q.reshape(_NBUF, _CJ, _IPW, _D)
    wid = lax.axis_index("s") * _NC + lax.axis_index("c")
    i0 = wid * _IPW
    pltpu.sync_copy(cue_hbm.at[pl.ds(i0, _IPW)], cue_v)

    lanes = lax.iota(jnp.int32, 16)

    def pack(v, carry):
        t = v * 16 + lanes
        col = lax.shift_right_logical(t, 5)
        row4 = lax.shift_left(lax.bitwise_and(t, _IQ - 1), 2)
        g0_ = plsc.load_gather(cue_v, [row4, col])
        g1_ = plsc.load_gather(cue_v, [row4 + 1, col])
        g2_ = plsc.load_gather(cue_v, [row4 + 2, col])
        g3_ = plsc.load_gather(cue_v, [row4 + 3, col])
        q = ((g0_ * 4 + g1_) * 4 + g2_) * 4 + g3_
        idx_v[pl.ds(pl.multiple_of(v * 16, 16), 16)] = q
        return carry

    lax.fori_loop(0, _NVEC, pack, 0)

    gsems = (g0, g1, g2)
    osems = (o0, o1, o2)

    def start_gather(c):
        b = c % _NBUF
        return pltpu.async_copy(
            qtable_hbm.at[idx_v.at[pl.ds(c * _CH, _CH)]], rows_q.at[b],
            gsems[b])

    gather_pending = [None] * _NBUF
    out_pending = [None] * _NBUF
    for c in range(min(2, _NCHUNK)):
        gather_pending[c % _NBUF] = start_gather(c)
    for c in range(_NCHUNK):
        b = c % _NBUF
        gather_pending[b].wait()
        out_pending[b] = pltpu.async_copy(
            rows_t.at[b],
            out_hbm.at[pl.ds(c * _CJ, _CJ), pl.ds(i0, _IPW)], osems[b])
        n = c + 2
        if n < _NCHUNK:
            bn = n % _NBUF
            if out_pending[bn] is not None:
                out_pending[bn].wait()
                out_pending[bn] = None
            gather_pending[bn] = start_gather(n)
    for b in range(_NBUF):
        if out_pending[b] is not None:
            out_pending[b].wait()


def kernel(cue, table):
    digits = (jnp.arange(256, dtype=jnp.int32)[:, None]
              // jnp.array([64, 16, 4, 1], dtype=jnp.int32)) % 4
    qtable = jnp.take(table.astype(jnp.float32), digits, axis=0)
    out_t = _embed_sc(cue.astype(jnp.int32), qtable)
    return jnp.transpose(out_t, (1, 0, 2))


# speedup vs baseline: 2.1189x; 1.1310x over previous
import functools

import jax
import jax.numpy as jnp
from jax import lax
from jax.experimental import pallas as pl
from jax.experimental.pallas import tpu as pltpu
from jax.experimental.pallas import tpu_sc as plsc

_N_ROWS = 4096
_N_COLS = 50
_D = 128
_NC = 2
_NS = 16
_NW = _NC * _NS
_IPW = _N_ROWS // _NW
_QPW = _IPW * _N_COLS // 4
_NVEC = _QPW // 16
_IQ = _IPW // 4
_CJ = 2
_CH = _CJ * _IQ
_NCHUNK = _N_COLS // _CJ
_NBUF = 3

_mesh = plsc.VectorSubcoreMesh(core_axis_name="c", subcore_axis_name="s")


@functools.partial(
    pl.kernel,
    mesh=_mesh,
    compiler_params=pltpu.CompilerParams(needs_layout_passes=False),
    out_type=(
        jax.ShapeDtypeStruct((_N_COLS, _N_ROWS, _D), jnp.float32),
        jax.ShapeDtypeStruct((_NC * 256, 4, _D), jnp.float32),
    ),
    scratch_types=[
        pltpu.VMEM((4, _D), jnp.float32),
        pltpu.VMEM((16, 4, _D), jnp.float32),
        pltpu.VMEM((_IPW, _N_COLS), jnp.int32),
        pltpu.VMEM((_QPW,), jnp.int32),
        pltpu.VMEM((_NBUF, _CH, 4, _D), jnp.float32),
        pltpu.SemaphoreType.DMA,
        pltpu.SemaphoreType.DMA,
        pltpu.SemaphoreType.DMA,
        pltpu.SemaphoreType.DMA,
        pltpu.SemaphoreType.DMA,
        pltpu.SemaphoreType.DMA,
    ],
)
def _embed_sc(cue_hbm, table_hbm, out_hbm, qtable_hbm, table_v, build_v,
              cue_v, idx_v, rows_q, g0, g1, g2, o0, o1, o2):
    rows_t = rows_q.reshape(_NBUF, _CJ, _IPW, _D)
    cid = lax.axis_index("c")
    sid = lax.axis_index("s")
    wid = sid * _NC + cid
    i0 = wid * _IPW

    pltpu.sync_copy(table_hbm, table_v)
    pltpu.sync_copy(cue_hbm.at[pl.ds(i0, _IPW)], cue_v)
    lanes = lax.iota(jnp.int32, 16)
    for j in range(16):
        p = sid * 16 + j
        for k in range(4):
            d = lax.shift_right_logical(p, 2 * (3 - k)) & 3
            d_vec = d + lanes * 0
            for c8 in range(_D // 16):
                val = plsc.load_gather(table_v, [d_vec, c8 * 16 + lanes])
                build_v[j, k, pl.ds(c8 * 16, 16)] = val
    qrow0 = cid * 256 + sid * 16
    pltpu.sync_copy(build_v, qtable_hbm.at[pl.ds(qrow0, 16)])
    plsc.subcore_barrier()

    bias = cid * 256

    def pack(v, carry):
        t = v * 16 + lanes
        col = lax.shift_right_logical(t, 5)
        row4 = lax.shift_left(lax.bitwise_and(t, _IQ - 1), 2)
        g0_ = plsc.load_gather(cue_v, [row4, col])
        g1_ = plsc.load_gather(cue_v, [row4 + 1, col])
        g2_ = plsc.load_gather(cue_v, [row4 + 2, col])
        g3_ = plsc.load_gather(cue_v, [row4 + 3, col])
        q = ((g0_ * 4 + g1_) * 4 + g2_) * 4 + g3_ + bias
        idx_v[pl.ds(pl.multiple_of(v * 16, 16), 16)] = q
        return carry

    lax.fori_loop(0, _NVEC, pack, 0)

    gsems = (g0, g1, g2)
    osems = (o0, o1, o2)

    def start_gather(c):
        b = c % _NBUF
        return pltpu.async_copy(
            qtable_hbm.at[idx_v.at[pl.ds(c * _CH, _CH)]], rows_q.at[b],
            gsems[b])

    gather_pending = [None] * _NBUF
    out_pending = [None] * _NBUF
    for c in range(min(2, _NCHUNK)):
        gather_pending[c % _NBUF] = start_gather(c)
    for c in range(_NCHUNK):
        b = c % _NBUF
        gather_pending[b].wait()
        out_pending[b] = pltpu.async_copy(
            rows_t.at[b],
            out_hbm.at[pl.ds(c * _CJ, _CJ), pl.ds(i0, _IPW)], osems[b])
        n = c + 2
        if n < _NCHUNK:
            bn = n % _NBUF
            if out_pending[bn] is not None:
                out_pending[bn].wait()
                out_pending[bn] = None
            gather_pending[bn] = start_gather(n)
    for b in range(_NBUF):
        if out_pending[b] is not None:
            out_pending[b].wait()


def kernel(cue, table):
    out_t, _ = _embed_sc(cue.astype(jnp.int32), table.astype(jnp.float32))
    return jnp.transpose(out_t, (1, 0, 2))


# speedup vs baseline: 3.6533x; 1.7241x over previous
import functools

import jax
import jax.numpy as jnp
from jax import lax
from jax.experimental import pallas as pl
from jax.experimental.pallas import tpu as pltpu
from jax.experimental.pallas import tpu_sc as plsc

_N_ROWS = 4096
_N_COLS = 50
_D = 128
_NC = 2
_NS = 16
_NW = _NC * _NS
_IPW = _N_ROWS // _NW
_QPW = _IPW * _N_COLS // 4
_NVEC = _QPW // 16
_IQ = _IPW // 4
_CJ = 2
_CH = _CJ * _IQ
_NCHUNK = _N_COLS // _CJ
_NBUF = 2

_mesh = plsc.VectorSubcoreMesh(core_axis_name="c", subcore_axis_name="s")


@functools.partial(
    pl.kernel,
    mesh=_mesh,
    compiler_params=pltpu.CompilerParams(needs_layout_passes=False),
    out_type=jax.ShapeDtypeStruct((_N_COLS, _N_ROWS, _D), jnp.float32),
    scratch_types=[
        pltpu.VMEM_SHARED((256, 4, _D), jnp.float32),
        pltpu.VMEM((4, _D), jnp.float32),
        pltpu.VMEM((16, 4, _D), jnp.float32),
        pltpu.VMEM((_IPW, _N_COLS), jnp.int32),
        pltpu.VMEM((_QPW,), jnp.int32),
        pltpu.VMEM((_NBUF, _CH, 4, _D), jnp.float32),
        pltpu.SemaphoreType.DMA,
        pltpu.SemaphoreType.DMA,
        pltpu.SemaphoreType.DMA,
        pltpu.SemaphoreType.DMA,
    ],
)
def _embed_sc(cue_hbm, table_hbm, out_hbm, qtable_sh, table_v, build_v,
              cue_v, idx_v, rows_q, g0, g1, o0, o1):
    rows_t = rows_q.reshape(_NBUF, _CJ, _IPW, _D)
    cid = lax.axis_index("c")
    sid = lax.axis_index("s")
    wid = sid * _NC + cid
    i0 = wid * _IPW

    pltpu.sync_copy(table_hbm, table_v)
    pltpu.sync_copy(cue_hbm.at[pl.ds(i0, _IPW)], cue_v)
    lanes = lax.iota(jnp.int32, 16)
    for j in range(16):
        p = sid * 16 + j
        for k in range(4):
            d = lax.shift_right_logical(p, 2 * (3 - k)) & 3
            d_vec = d + lanes * 0
            for c8 in range(_D // 16):
                val = plsc.load_gather(table_v, [d_vec, c8 * 16 + lanes])
                build_v[j, k, pl.ds(c8 * 16, 16)] = val
    pltpu.sync_copy(build_v, qtable_sh.at[pl.ds(sid * 16, 16)])
    plsc.subcore_barrier()


    def pack(v, carry):
        t = v * 16 + lanes
        col = lax.shift_right_logical(t, 5)
        row4 = lax.shift_left(lax.bitwise_and(t, _IQ - 1), 2)
        g0_ = plsc.load_gather(cue_v, [row4, col])
        g1_ = plsc.load_gather(cue_v, [row4 + 1, col])
        g2_ = plsc.load_gather(cue_v, [row4 + 2, col])
        g3_ = plsc.load_gather(cue_v, [row4 + 3, col])
        q = ((g0_ * 4 + g1_) * 4 + g2_) * 4 + g3_
        idx_v[pl.ds(pl.multiple_of(v * 16, 16), 16)] = q
        return carry

    lax.fori_loop(0, _NVEC, pack, 0)

    gsems = (g0, g1)
    osems = (o0, o1)

    def start_gather(c):
        b = c % _NBUF
        return pltpu.async_copy(
            qtable_sh.at[idx_v.at[pl.ds(c * _CH, _CH)]], rows_q.at[b],
            gsems[b])

    gather_pending = [None] * _NBUF
    out_pending = [None] * _NBUF
    gather_pending[0] = start_gather(0)
    for c in range(_NCHUNK):
        b = c % _NBUF
        gather_pending[b].wait()
        out_pending[b] = pltpu.async_copy(
            rows_t.at[b],
            out_hbm.at[pl.ds(c * _CJ, _CJ), pl.ds(i0, _IPW)], osems[b])
        n = c + 1
        if n < _NCHUNK:
            bn = n % _NBUF
            if out_pending[bn] is not None:
                out_pending[bn].wait()
                out_pending[bn] = None
            gather_pending[bn] = start_gather(n)
    for b in range(_NBUF):
        if out_pending[b] is not None:
            out_pending[b].wait()


def kernel(cue, table):
    out_t = _embed_sc(cue.astype(jnp.int32), table.astype(jnp.float32))
    return jnp.transpose(out_t, (1, 0, 2))
